# Initial kernel scaffold; baseline (speedup 1.0000x reference)
import jax
import jax.numpy as jnp
from jax.experimental import pallas as pl


def kernel(x, g, e):
    raise NotImplementedError("write your pallas kernel here")


# speedup vs baseline: 1.1088x; 1.1088x over previous
import functools

import jax
import jax.numpy as jnp
from jax import lax
from jax.experimental import pallas as pl
from jax.experimental.pallas import tpu as pltpu
from jax.experimental.pallas import tpu_sc as plsc

N, C, H, W = 4, 96, 224, 224
B = N * H * W
HW = H * W
NC, NS, L = 2, 16, 16
NW = NC * NS
BPT = B // NW
K = 64
NCHUNK = BPT // K


def _grid_kernel(table, gx_hbm, gy_hbm, out_hbm,
                 gx_v, gy_v, idx_v, w_v, rows_v, out_v, gsem):
    wid = lax.axis_index("s") * NC + lax.axis_index("c")
    base = wid * BPT
    n_base = (wid // (HW // BPT)) * HW

    pltpu.sync_copy(gx_hbm.at[pl.ds(base, BPT)], gx_v)
    pltpu.sync_copy(gy_hbm.at[pl.ds(base, BPT)], gy_v)

    def prime(cidx, slot):
        for i in range(K // L):
            s = pl.ds(i * L, L)
            gx = gx_v[pl.ds(cidx * K + i * L, L)]
            gy = gy_v[pl.ds(cidx * K + i * L, L)]
            ix = (gx + 1.0) * 0.5 * (W - 1)
            iy = (gy + 1.0) * 0.5 * (H - 1)
            ix0 = ix.astype(jnp.int32)
            iy0 = iy.astype(jnp.int32)
            wx1 = ix - ix0.astype(jnp.float32)
            wy1 = iy - iy0.astype(jnp.float32)
            wx0 = 1.0 - wx1
            wy0 = 1.0 - wy1
            ix1 = jnp.minimum(ix0 + 1, W - 1)
            iy1 = jnp.minimum(iy0 + 1, H - 1)
            r0 = n_base + iy0 * W
            r1 = n_base + iy1 * W
            idx_v[slot, 0, s] = r0 + ix0
            idx_v[slot, 1, s] = r0 + ix1
            idx_v[slot, 2, s] = r1 + ix0
            idx_v[slot, 3, s] = r1 + ix1
            w_v[slot, 0, s] = wy0 * wx0
            w_v[slot, 1, s] = wy0 * wx1
            w_v[slot, 2, s] = wy1 * wx0
            w_v[slot, 3, s] = wy1 * wx1

    def fire(slot):
        return [pltpu.async_copy(table.at[idx_v.at[slot, c]],
                                 rows_v.at[slot, c], gsem)
                for c in range(4)]

    def blend(slot):
        def gbody(gi, _):
            wv = [w_v[slot, c, pl.ds(gi * L, L)] for c in range(4)]
            for jj in range(L):
                j = gi * L + jj
                w00 = jnp.full((L,), wv[0][jj], jnp.float32)
                w01 = jnp.full((L,), wv[1][jj], jnp.float32)
                w10 = jnp.full((L,), wv[2][jj], jnp.float32)
                w11 = jnp.full((L,), wv[3][jj], jnp.float32)
                for cg in range(C // L):
                    cs = pl.ds(cg * L, L)
                    out_v[slot, j, cs] = (w00 * rows_v[slot, 0, j, cs]
                                          + w01 * rows_v[slot, 1, j, cs]
                                          + w10 * rows_v[slot, 2, j, cs]
                                          + w11 * rows_v[slot, 3, j, cs])
            return 0
        lax.fori_loop(0, K // L, gbody, 0)

    def chunk_body(cidx, _):
        prime(cidx, 0)
        for cp in fire(0):
            cp.wait()
        blend(0)
        pltpu.sync_copy(out_v.at[0], out_hbm.at[pl.ds(base + cidx * K, K)])
        return 0

    lax.fori_loop(0, NCHUNK, chunk_body, 0)


_grid_call = functools.partial(
    pl.kernel,
    out_type=jax.ShapeDtypeStruct((B, C), jnp.float32),
    mesh=plsc.VectorSubcoreMesh(core_axis_name="c", subcore_axis_name="s"),
    scratch_types=[
        pltpu.VMEM((BPT,), jnp.float32),
        pltpu.VMEM((BPT,), jnp.float32),
        pltpu.VMEM((2, 4, K), jnp.int32),
        pltpu.VMEM((2, 4, K), jnp.float32),
        pltpu.VMEM((2, 4, K, C), jnp.float32),
        pltpu.VMEM((2, K, C), jnp.float32),
        pltpu.SemaphoreType.DMA,
    ],
    compiler_params=pltpu.CompilerParams(use_tc_tiling_on_sc=False),
)(_grid_kernel)


def kernel(x, g, e):
    del e
    table = x.transpose(0, 2, 3, 1).reshape(B, C)
    gflat = g.reshape(B, 2)
    out = _grid_call(table, gflat[:, 0], gflat[:, 1])
    return out.reshape(N, H, W, C).transpose(0, 3, 1, 2)


# speedup vs baseline: 1.3518x; 1.2192x over previous
import functools

import jax
import jax.numpy as jnp
from jax import lax
from jax.experimental import pallas as pl
from jax.experimental.pallas import tpu as pltpu
from jax.experimental.pallas import tpu_sc as plsc

N, C, H, W = 4, 96, 224, 224
B = N * H * W
HW = H * W
NC, NS, L = 2, 16, 16
NW = NC * NS
BPT = B // NW
K = 64
NCHUNK = BPT // K


def _grid_kernel(table, gx_hbm, gy_hbm, out_hbm,
                 gx_v, gy_v, idx_v, w_v, rows_v, out_v,
                 gsem0, gsem1, ssem0, ssem1):
    wid = lax.axis_index("s") * NC + lax.axis_index("c")
    base = wid * BPT
    n_base = (wid // (HW // BPT)) * HW

    pltpu.sync_copy(gx_hbm.at[pl.ds(base, BPT)], gx_v)
    pltpu.sync_copy(gy_hbm.at[pl.ds(base, BPT)], gy_v)

    def prime(cidx, slot):
        for i in range(K // L):
            s = pl.ds(i * L, L)
            gx = gx_v[pl.ds(cidx * K + i * L, L)]
            gy = gy_v[pl.ds(cidx * K + i * L, L)]
            ix = (gx + 1.0) * 0.5 * (W - 1)
            iy = (gy + 1.0) * 0.5 * (H - 1)
            ix0 = ix.astype(jnp.int32)
            iy0 = iy.astype(jnp.int32)
            wx1 = ix - ix0.astype(jnp.float32)
            wy1 = iy - iy0.astype(jnp.float32)
            wx0 = 1.0 - wx1
            wy0 = 1.0 - wy1
            ix1 = jnp.minimum(ix0 + 1, W - 1)
            iy1 = jnp.minimum(iy0 + 1, H - 1)
            r0 = n_base + iy0 * W
            r1 = n_base + iy1 * W
            idx_v[slot, 0, s] = r0 + ix0
            idx_v[slot, 1, s] = r0 + ix1
            idx_v[slot, 2, s] = r1 + ix0
            idx_v[slot, 3, s] = r1 + ix1
            w_v[slot, 0, s] = wy0 * wx0
            w_v[slot, 1, s] = wy0 * wx1
            w_v[slot, 2, s] = wy1 * wx0
            w_v[slot, 3, s] = wy1 * wx1

    def fire(cidx, slot, sem):
        prime(cidx, slot)
        for c in range(4):
            pltpu.make_async_copy(table.at[idx_v.at[slot, c]],
                                  rows_v.at[slot, c], sem).start()

    def drain_gather(slot, sem):
        for c in range(4):
            pltpu.make_async_copy(table.at[idx_v.at[slot, c]],
                                  rows_v.at[slot, c], sem).wait()

    def fire_scatter(cidx, slot, sem):
        pltpu.make_async_copy(out_v.at[slot],
                              out_hbm.at[pl.ds(base + cidx * K, K)],
                              sem).start()

    def drain_scatter(cidx, slot, sem):
        pltpu.make_async_copy(out_v.at[slot],
                              out_hbm.at[pl.ds(base + cidx * K, K)],
                              sem).wait()

    def blend(slot):
        def gbody(gi, _):
            wv = [w_v[slot, c, pl.ds(gi * L, L)] for c in range(4)]
            for jj in range(L):
                j = gi * L + jj
                w00 = jnp.full((L,), wv[0][jj], jnp.float32)
                w01 = jnp.full((L,), wv[1][jj], jnp.float32)
                w10 = jnp.full((L,), wv[2][jj], jnp.float32)
                w11 = jnp.full((L,), wv[3][jj], jnp.float32)
                for cg in range(C // L):
                    cs = pl.ds(cg * L, L)
                    out_v[slot, j, cs] = (w00 * rows_v[slot, 0, j, cs]
                                          + w01 * rows_v[slot, 1, j, cs]
                                          + w10 * rows_v[slot, 2, j, cs]
                                          + w11 * rows_v[slot, 3, j, cs])
            return 0
        lax.fori_loop(0, K // L, gbody, 0)

    fire(0, 0, gsem0)
    fire(1, 1, gsem1)

    def step_body(step, _):
        for b, gs, ss in ((0, gsem0, ssem0), (1, gsem1, ssem1)):
            c = step * 2 + b
            drain_gather(b, gs)

            @pl.when(step >= 1)
            def _():
                drain_scatter(c - 2, b, ss)

            blend(b)
            fire_scatter(c, b, ss)

            @pl.when(step < NCHUNK // 2 - 1)
            def _():
                fire(c + 2, b, gs)
        return 0

    lax.fori_loop(0, NCHUNK // 2, step_body, 0)
    drain_scatter(NCHUNK - 2, 0, ssem0)
    drain_scatter(NCHUNK - 1, 1, ssem1)


_grid_call = functools.partial(
    pl.kernel,
    out_type=jax.ShapeDtypeStruct((B, C), jnp.float32),
    mesh=plsc.VectorSubcoreMesh(core_axis_name="c", subcore_axis_name="s"),
    scratch_types=[
        pltpu.VMEM((BPT,), jnp.float32),
        pltpu.VMEM((BPT,), jnp.float32),
        pltpu.VMEM((2, 4, K), jnp.int32),
        pltpu.VMEM((2, 4, K), jnp.float32),
        pltpu.VMEM((2, 4, K, C), jnp.float32),
        pltpu.VMEM((2, K, C), jnp.float32),
        pltpu.SemaphoreType.DMA,
        pltpu.SemaphoreType.DMA,
        pltpu.SemaphoreType.DMA,
        pltpu.SemaphoreType.DMA,
    ],
    compiler_params=pltpu.CompilerParams(use_tc_tiling_on_sc=False),
)(_grid_kernel)


def kernel(x, g, e):
    del e
    table = x.transpose(0, 2, 3, 1).reshape(B, C)
    gflat = g.reshape(B, 2)
    out = _grid_call(table, gflat[:, 0], gflat[:, 1])
    return out.reshape(N, H, W, C).transpose(0, 3, 1, 2)
